# Initial kernel scaffold; baseline (speedup 1.0000x reference)
import jax
import jax.numpy as jnp
from jax.experimental import pallas as pl


def kernel(x, edge_index, edge_weight, W1, W2, W3):
    raise NotImplementedError("write your pallas kernel here")


# speedup vs baseline: 6.0736x; 6.0736x over previous
import functools

import jax
import jax.numpy as jnp
from jax import lax
from jax.experimental import pallas as pl
from jax.experimental.pallas import tpu as pltpu
from jax.experimental.pallas import tpu_sc as plsc

N = 10000
E = 320000
F = 128
H1 = 32
H2 = 16

NC = 2
NS = 16
NW = NC * NS
SUB = 125
EW = E // NW
NSUBW = EW // SUB
ROWS_T = 624
TAIL = N - NS * ROWS_T


def _make_spmm():
    mesh = plsc.VectorSubcoreMesh(core_axis_name="c", subcore_axis_name="s")

    @functools.partial(
        pl.kernel,
        out_type=jax.ShapeDtypeStruct((NC, N, H1), jnp.float32),
        mesh=mesh,
        scratch_types=[
            pltpu.VMEM((NSUBW, SUB), jnp.int32),
            pltpu.VMEM((NSUBW, SUB), jnp.int32),
            pltpu.VMEM((EW,), jnp.float32),
            pltpu.VMEM((SUB, H1), jnp.float32),
            pltpu.VMEM((SUB, H1), jnp.float32),
            pltpu.VMEM_SHARED((N, H1), jnp.float32),
            pltpu.SemaphoreType.DMA,
        ],
        compiler_params=pltpu.CompilerParams(use_tc_tiling_on_sc=False,
                                             needs_layout_passes=False),
    )
    def spmm(src_hbm, dst_hbm, w_hbm, x_hbm, zero_hbm, out_hbm,
             src_v, dst_v, w_v, g_v, s_v, acc, sem):
        cid = lax.axis_index("c")
        sid = lax.axis_index("s")
        wid = cid * NS + sid
        base = wid * NSUBW

        pltpu.sync_copy(src_hbm.at[pl.ds(base, NSUBW)], src_v)
        pltpu.sync_copy(dst_hbm.at[pl.ds(base, NSUBW)], dst_v)
        pltpu.sync_copy(w_hbm.at[pl.ds(wid * EW, EW)], w_v)
        pltpu.sync_copy(zero_hbm.at[pl.ds(sid * ROWS_T, ROWS_T)],
                        acc.at[pl.ds(sid * ROWS_T, ROWS_T)])

        @pl.when(sid == NS - 1)
        def _():
            pltpu.sync_copy(zero_hbm.at[pl.ds(NS * ROWS_T, TAIL)],
                            acc.at[pl.ds(NS * ROWS_T, TAIL)])

        plsc.subcore_barrier()

        def sub_body(jj, carry):
            pltpu.async_copy(x_hbm.at[src_v.at[jj]], g_v, sem).wait()

            wbase = jj * SUB

            def mul_body(e, c2):
                wv = plsc.load_gather(w_v, [jnp.broadcast_to(wbase + e, (16,))])
                s_v[e, 0:16] = g_v[e, 0:16] * wv
                s_v[e, 16:32] = g_v[e, 16:32] * wv
                return c2

            lax.fori_loop(0, SUB, mul_body, 0)

            pltpu.sync_copy(s_v, acc.at[dst_v.at[jj]], add=True)
            return carry

        lax.fori_loop(0, NSUBW, sub_body, 0)
        plsc.subcore_barrier()

        pltpu.sync_copy(acc.at[pl.ds(sid * ROWS_T, ROWS_T)],
                        out_hbm.at[cid, pl.ds(sid * ROWS_T, ROWS_T)])

        @pl.when(sid == NS - 1)
        def _():
            pltpu.sync_copy(acc.at[pl.ds(NS * ROWS_T, TAIL)],
                            out_hbm.at[cid, pl.ds(NS * ROWS_T, TAIL)])

    return spmm


_spmm = _make_spmm()


def _mm_x_w1(x, W1):
    BM = 1000

    def body(x_ref, w_ref, o_ref):
        o_ref[...] = lax.dot_general(
            x_ref[...], w_ref[...], (((1,), (0,)), ((), ())),
            preferred_element_type=jnp.float32)

    return pl.pallas_call(
        body,
        grid=(N // BM,),
        in_specs=[pl.BlockSpec((BM, F), lambda i: (i, 0)),
                  pl.BlockSpec((F, H1), lambda i: (0, 0))],
        out_specs=pl.BlockSpec((BM, H1), lambda i: (i, 0)),
        out_shape=jax.ShapeDtypeStruct((N, H1), jnp.float32),
    )(x, W1)


def _hidden_mm(parts, W23):
    BM = 1000

    def body(p_ref, w_ref, o_ref):
        h = jnp.maximum(p_ref[0] + p_ref[1], 0.0)
        o_ref[...] = lax.dot_general(
            h, w_ref[...], (((1,), (0,)), ((), ())),
            preferred_element_type=jnp.float32)

    return pl.pallas_call(
        body,
        grid=(N // BM,),
        in_specs=[pl.BlockSpec((NC, BM, H1), lambda i: (0, i, 0)),
                  pl.BlockSpec((H1, 2 * H2), lambda i: (0, 0))],
        out_specs=pl.BlockSpec((BM, 2 * H2), lambda i: (i, 0)),
        out_shape=jax.ShapeDtypeStruct((N, 2 * H2), jnp.float32),
    )(parts, W23)


def _make_z(parts, noise):
    BM = 1000

    def body(p_ref, n_ref, o_ref):
        s = p_ref[0] + p_ref[1]
        o_ref[...] = s[:, :H2] + n_ref[...] * jnp.exp(s[:, H2:])

    return pl.pallas_call(
        body,
        grid=(N // BM,),
        in_specs=[pl.BlockSpec((NC, BM, 2 * H2), lambda i: (0, i, 0)),
                  pl.BlockSpec((BM, H2), lambda i: (i, 0))],
        out_specs=pl.BlockSpec((BM, H2), lambda i: (i, 0)),
        out_shape=jax.ShapeDtypeStruct((N, H2), jnp.float32),
    )(parts, noise)


def _decoder(z):
    BM = 400

    def body(zi_ref, zj_ref, o_ref):
        o_ref[...] = lax.dot_general(
            zi_ref[...], zj_ref[...], (((1,), (1,)), ((), ())),
            preferred_element_type=jnp.float32)

    return pl.pallas_call(
        body,
        grid=(N // BM,),
        in_specs=[pl.BlockSpec((BM, H2), lambda i: (i, 0)),
                  pl.BlockSpec((N, H2), lambda i: (0, 0))],
        out_specs=pl.BlockSpec((BM, N), lambda i: (i, 0)),
        out_shape=jax.ShapeDtypeStruct((N, N), jnp.float32),
    )(z, z)


def kernel(x, edge_index, edge_weight, W1, W2, W3):
    src2d = jnp.reshape(edge_index[0], (E // SUB, SUB))
    dst2d = jnp.reshape(edge_index[1], (E // SUB, SUB))
    zeros = jnp.zeros((N, H1), dtype=jnp.float32)
    W23 = jnp.concatenate([W2, W3], axis=1)
    noise = jax.random.normal(jax.random.key(42), (N, H2), dtype=jnp.float32)

    h0 = _mm_x_w1(x, W1)
    parts1 = _spmm(src2d, dst2d, edge_weight,h0, zeros)
    m = _hidden_mm(parts1, W23)
    parts2 = _spmm(src2d, dst2d, edge_weight,m, zeros)
    z = _make_z(parts2, noise)
    return jnp.reshape(_decoder(z), (-1,))


# speedup vs baseline: 6.6865x; 1.1009x over previous
import functools

import numpy as np
import jax
import jax.numpy as jnp
from jax import lax
from jax.experimental import pallas as pl
from jax.experimental.pallas import tpu as pltpu
from jax.experimental.pallas import tpu_sc as plsc

N = 10000
E = 320000
F = 128
H1 = 32
H2 = 16

NC = 2
NS = 16
NW = NC * NS
SUB = 125
EW = E // NW
NSUBW = EW // SUB
ROWS_T = 624
TAIL = N - NS * ROWS_T


def _make_spmm():
    mesh = plsc.VectorSubcoreMesh(core_axis_name="c", subcore_axis_name="s")

    @functools.partial(
        pl.kernel,
        out_type=jax.ShapeDtypeStruct((NC, N, H1), jnp.float32),
        mesh=mesh,
        scratch_types=[
            pltpu.VMEM((NSUBW, SUB), jnp.int32),
            pltpu.VMEM((NSUBW, SUB), jnp.int32),
            pltpu.VMEM((EW,), jnp.float32),
            pltpu.VMEM((2, SUB, H1), jnp.float32),
            pltpu.VMEM((2, SUB, H1), jnp.float32),
            pltpu.VMEM_SHARED((N, H1), jnp.float32),
            pltpu.SemaphoreType.DMA,
            pltpu.SemaphoreType.DMA,
        ],
        compiler_params=pltpu.CompilerParams(use_tc_tiling_on_sc=False,
                                             needs_layout_passes=False),
    )
    def spmm(src_hbm, dst_hbm, w_hbm, x_hbm, zero_hbm, out_hbm,
             src_v, dst_v, w_v, g_v, s_v, acc, sem_g, sem_s):
        cid = lax.axis_index("c")
        sid = lax.axis_index("s")
        wid = cid * NS + sid
        base = wid * NSUBW

        pltpu.sync_copy(src_hbm.at[pl.ds(base, NSUBW)], src_v)
        pltpu.sync_copy(dst_hbm.at[pl.ds(base, NSUBW)], dst_v)
        pltpu.sync_copy(w_hbm.at[pl.ds(wid * EW, EW)], w_v)
        pltpu.sync_copy(zero_hbm.at[pl.ds(sid * ROWS_T, ROWS_T)],
                        acc.at[pl.ds(sid * ROWS_T, ROWS_T)])

        @pl.when(sid == NS - 1)
        def _():
            pltpu.sync_copy(zero_hbm.at[pl.ds(NS * ROWS_T, TAIL)],
                            acc.at[pl.ds(NS * ROWS_T, TAIL)])

        plsc.subcore_barrier()

        pltpu.async_copy(x_hbm.at[src_v.at[0]], g_v.at[0], sem_g)

        def outer(t, carry):
            for b in range(2):
                j = 2 * t + b
                pltpu.make_async_copy(zero_hbm.at[pl.ds(0, SUB)],
                                      g_v.at[b], sem_g).wait()

                @pl.when(j + 1 < NSUBW)
                def _():
                    pltpu.async_copy(x_hbm.at[src_v.at[j + 1]],
                                     g_v.at[1 - b], sem_g)

                @pl.when(t >= 1)
                def _():
                    pltpu.make_async_copy(zero_hbm.at[pl.ds(0, SUB)],
                                          s_v.at[b], sem_s).wait()

                wbase = j * SUB

                def mul_body(e, c2):
                    wv = plsc.load_gather(
                        w_v, [jnp.broadcast_to(wbase + e, (16,))])
                    s_v[b, e, 0:16] = g_v[b, e, 0:16] * wv
                    s_v[b, e, 16:32] = g_v[b, e, 16:32] * wv
                    return c2

                lax.fori_loop(0, SUB, mul_body, 0, unroll=5)

                pltpu.async_copy(s_v.at[b], acc.at[dst_v.at[j]], sem_s,
                                 add=True)
            return carry

        lax.fori_loop(0, NSUBW // 2, outer, 0)

        for b in range(2):
            pltpu.make_async_copy(zero_hbm.at[pl.ds(0, SUB)],
                                  s_v.at[b], sem_s).wait()
        plsc.subcore_barrier()

        pltpu.sync_copy(acc.at[pl.ds(sid * ROWS_T, ROWS_T)],
                        out_hbm.at[cid, pl.ds(sid * ROWS_T, ROWS_T)])

        @pl.when(sid == NS - 1)
        def _():
            pltpu.sync_copy(acc.at[pl.ds(NS * ROWS_T, TAIL)],
                            out_hbm.at[cid, pl.ds(NS * ROWS_T, TAIL)])

    return spmm


_spmm = _make_spmm()

_ZEROS = np.zeros((N, H1), dtype=np.float32)


def _mm_x_w1(x, W1):
    BM = 1000

    def body(x_ref, w_ref, o_ref):
        o_ref[...] = lax.dot_general(
            x_ref[...], w_ref[...], (((1,), (0,)), ((), ())),
            preferred_element_type=jnp.float32)

    return pl.pallas_call(
        body,
        grid=(N // BM,),
        in_specs=[pl.BlockSpec((BM, F), lambda i: (i, 0)),
                  pl.BlockSpec((F, H1), lambda i: (0, 0))],
        out_specs=pl.BlockSpec((BM, H1), lambda i: (i, 0)),
        out_shape=jax.ShapeDtypeStruct((N, H1), jnp.float32),
    )(x, W1)


def _hidden_mm(parts, W23):
    BM = 1000

    def body(p_ref, w_ref, o_ref):
        h = jnp.maximum(p_ref[0] + p_ref[1], 0.0)
        o_ref[...] = lax.dot_general(
            h, w_ref[...], (((1,), (0,)), ((), ())),
            preferred_element_type=jnp.float32)

    return pl.pallas_call(
        body,
        grid=(N // BM,),
        in_specs=[pl.BlockSpec((NC, BM, H1), lambda i: (0, i, 0)),
                  pl.BlockSpec((H1, 2 * H2), lambda i: (0, 0))],
        out_specs=pl.BlockSpec((BM, 2 * H2), lambda i: (i, 0)),
        out_shape=jax.ShapeDtypeStruct((N, 2 * H2), jnp.float32),
    )(parts, W23)


def _make_z(parts, noise):
    BM = 1000

    def body(p_ref, n_ref, o_ref):
        s = p_ref[0] + p_ref[1]
        o_ref[...] = s[:, :H2] + n_ref[...] * jnp.exp(s[:, H2:])

    return pl.pallas_call(
        body,
        grid=(N // BM,),
        in_specs=[pl.BlockSpec((NC, BM, 2 * H2), lambda i: (0, i, 0)),
                  pl.BlockSpec((BM, H2), lambda i: (i, 0))],
        out_specs=pl.BlockSpec((BM, H2), lambda i: (i, 0)),
        out_shape=jax.ShapeDtypeStruct((N, H2), jnp.float32),
    )(parts, noise)


def _decoder(z):
    BM = 400

    def body(zi_ref, zj_ref, o_ref):
        o_ref[...] = lax.dot_general(
            zi_ref[...], zj_ref[...], (((1,), (1,)), ((), ())),
            preferred_element_type=jnp.float32)

    return pl.pallas_call(
        body,
        grid=(N // BM,),
        in_specs=[pl.BlockSpec((BM, H2), lambda i: (i, 0)),
                  pl.BlockSpec((N, H2), lambda i: (0, 0))],
        out_specs=pl.BlockSpec((BM, N), lambda i: (i, 0)),
        out_shape=jax.ShapeDtypeStruct((N, N), jnp.float32),
    )(z, z)


def kernel(x, edge_index, edge_weight, W1, W2, W3):
    src2d = jnp.reshape(edge_index[0], (E // SUB, SUB))
    dst2d = jnp.reshape(edge_index[1], (E // SUB, SUB))
    W23 = jnp.concatenate([W2, W3], axis=1)

    h0 = _mm_x_w1(x, W1)
    parts1 = _spmm(src2d, dst2d, edge_weight, h0, _ZEROS)
    m = _hidden_mm(parts1, W23)
    parts2 = _spmm(src2d, dst2d, edge_weight, m, _ZEROS)
    noise = jax.random.normal(jax.random.key(42), (N, H2), dtype=jnp.float32)
    z = _make_z(parts2, noise)
    return jnp.reshape(_decoder(z), (-1,))


# speedup vs baseline: 8.0142x; 1.1986x over previous
import functools

import numpy as np
import jax
import jax.numpy as jnp
from jax import lax
from jax.experimental import pallas as pl
from jax.experimental.pallas import tpu as pltpu
from jax.experimental.pallas import tpu_sc as plsc

N = 10000
E = 320000
F = 128
H1 = 32
H2 = 16

NC = 2
NS = 16
NW = NC * NS
SUB = 125
EW = E // NW
NSUBW = EW // SUB
RING = 4
ROWS_T = 624
TAIL = N - NS * ROWS_T


def _make_spmm():
    mesh = plsc.VectorSubcoreMesh(core_axis_name="c", subcore_axis_name="s")

    @functools.partial(
        pl.kernel,
        out_type=jax.ShapeDtypeStruct((NC, N, H1), jnp.float32),
        mesh=mesh,
        scratch_types=[
            pltpu.VMEM((NSUBW, SUB), jnp.int32),
            pltpu.VMEM((NSUBW, SUB), jnp.int32),
            pltpu.VMEM((EW,), jnp.float32),
            pltpu.VMEM((RING, SUB, H1), jnp.float32),
            pltpu.VMEM((RING, SUB, H1), jnp.float32),
            pltpu.VMEM_SHARED((N, H1), jnp.float32),
            pltpu.SemaphoreType.DMA,
            pltpu.SemaphoreType.DMA,
        ],
        compiler_params=pltpu.CompilerParams(use_tc_tiling_on_sc=False,
                                             needs_layout_passes=False),
    )
    def spmm(src_hbm, dst_hbm, w_hbm, x_hbm, zero_hbm, out_hbm,
             src_v, dst_v, w_v, g_v, s_v, acc, sem_g, sem_s):
        cid = lax.axis_index("c")
        sid = lax.axis_index("s")
        wid = cid * NS + sid
        base = wid * NSUBW

        pltpu.sync_copy(src_hbm.at[pl.ds(base, NSUBW)], src_v)
        pltpu.sync_copy(dst_hbm.at[pl.ds(base, NSUBW)], dst_v)
        pltpu.sync_copy(w_hbm.at[pl.ds(wid * EW, EW)], w_v)
        pltpu.sync_copy(zero_hbm.at[pl.ds(sid * ROWS_T, ROWS_T)],
                        acc.at[pl.ds(sid * ROWS_T, ROWS_T)])

        @pl.when(sid == NS - 1)
        def _():
            pltpu.sync_copy(zero_hbm.at[pl.ds(NS * ROWS_T, TAIL)],
                            acc.at[pl.ds(NS * ROWS_T, TAIL)])

        plsc.subcore_barrier()

        for p in range(RING - 1):
            pltpu.async_copy(x_hbm.at[src_v.at[p]], g_v.at[p], sem_g)

        def process(j, carry):
            b = lax.rem(j, RING)
            pltpu.make_async_copy(zero_hbm.at[pl.ds(0, SUB)],
                                  g_v.at[b], sem_g).wait()

            @pl.when(j >= RING)
            def _():
                pltpu.make_async_copy(zero_hbm.at[pl.ds(0, SUB)],
                                      s_v.at[b], sem_s).wait()

            wbase = j * SUB
            for g in range(SUB // 16):
                w16 = w_v[pl.ds(wbase + g * 16, 16)]
                for l in range(16):
                    e = g * 16 + l
                    wv = jnp.broadcast_to(w16[l], (16,))
                    s_v[b, e, 0:16] = g_v[b, e, 0:16] * wv
                    s_v[b, e, 16:32] = g_v[b, e, 16:32] * wv

            def mul_tail(e, c2):
                wv = plsc.load_gather(w_v, [jnp.broadcast_to(wbase + e, (16,))])
                s_v[b, e, 0:16] = g_v[b, e, 0:16] * wv
                s_v[b, e, 16:32] = g_v[b, e, 16:32] * wv
                return c2

            lax.fori_loop((SUB // 16) * 16, SUB, mul_tail, 0)

            pltpu.async_copy(s_v.at[b], acc.at[dst_v.at[j]], sem_s, add=True)

            @pl.when(j + RING - 1 < NSUBW)
            def _():
                pltpu.async_copy(x_hbm.at[src_v.at[j + RING - 1]],
                                 g_v.at[lax.rem(j + RING - 1, RING)], sem_g)
            return carry

        lax.fori_loop(0, NSUBW, process, 0)

        for b in range(RING):
            pltpu.make_async_copy(zero_hbm.at[pl.ds(0, SUB)],
                                  s_v.at[b], sem_s).wait()
        plsc.subcore_barrier()

        pltpu.sync_copy(acc.at[pl.ds(sid * ROWS_T, ROWS_T)],
                        out_hbm.at[cid, pl.ds(sid * ROWS_T, ROWS_T)])

        @pl.when(sid == NS - 1)
        def _():
            pltpu.sync_copy(acc.at[pl.ds(NS * ROWS_T, TAIL)],
                            out_hbm.at[cid, pl.ds(NS * ROWS_T, TAIL)])

    return spmm


_spmm = _make_spmm()

_ZEROS = np.zeros((N, H1), dtype=np.float32)


def _mm_x_w1(x, W1):
    BM = 1000

    def body(x_ref, w_ref, o_ref):
        o_ref[...] = lax.dot_general(
            x_ref[...], w_ref[...], (((1,), (0,)), ((), ())),
            preferred_element_type=jnp.float32)

    return pl.pallas_call(
        body,
        grid=(N // BM,),
        in_specs=[pl.BlockSpec((BM, F), lambda i: (i, 0)),
                  pl.BlockSpec((F, H1), lambda i: (0, 0))],
        out_specs=pl.BlockSpec((BM, H1), lambda i: (i, 0)),
        out_shape=jax.ShapeDtypeStruct((N, H1), jnp.float32),
    )(x, W1)


def _hidden_mm(parts, W23):
    BM = 1000

    def body(p_ref, w_ref, o_ref):
        h = jnp.maximum(p_ref[0] + p_ref[1], 0.0)
        o_ref[...] = lax.dot_general(
            h, w_ref[...], (((1,), (0,)), ((), ())),
            preferred_element_type=jnp.float32)

    return pl.pallas_call(
        body,
        grid=(N // BM,),
        in_specs=[pl.BlockSpec((NC, BM, H1), lambda i: (0, i, 0)),
                  pl.BlockSpec((H1, 2 * H2), lambda i: (0, 0))],
        out_specs=pl.BlockSpec((BM, 2 * H2), lambda i: (i, 0)),
        out_shape=jax.ShapeDtypeStruct((N, 2 * H2), jnp.float32),
    )(parts, W23)


def _make_z(parts, noise):
    BM = 1000

    def body(p_ref, n_ref, o_ref):
        s = p_ref[0] + p_ref[1]
        o_ref[...] = s[:, :H2] + n_ref[...] * jnp.exp(s[:, H2:])

    return pl.pallas_call(
        body,
        grid=(N // BM,),
        in_specs=[pl.BlockSpec((NC, BM, 2 * H2), lambda i: (0, i, 0)),
                  pl.BlockSpec((BM, H2), lambda i: (i, 0))],
        out_specs=pl.BlockSpec((BM, H2), lambda i: (i, 0)),
        out_shape=jax.ShapeDtypeStruct((N, H2), jnp.float32),
    )(parts, noise)


def _decoder(z):
    BM = 400

    def body(zi_ref, zj_ref, o_ref):
        o_ref[...] = lax.dot_general(
            zi_ref[...], zj_ref[...], (((1,), (1,)), ((), ())),
            preferred_element_type=jnp.float32)

    return pl.pallas_call(
        body,
        grid=(N // BM,),
        in_specs=[pl.BlockSpec((BM, H2), lambda i: (i, 0)),
                  pl.BlockSpec((N, H2), lambda i: (0, 0))],
        out_specs=pl.BlockSpec((BM, N), lambda i: (i, 0)),
        out_shape=jax.ShapeDtypeStruct((N, N), jnp.float32),
    )(z, z)


def kernel(x, edge_index, edge_weight, W1, W2, W3):
    src2d = jnp.reshape(edge_index[0], (E // SUB, SUB))
    dst2d = jnp.reshape(edge_index[1], (E // SUB, SUB))
    W23 = jnp.concatenate([W2, W3], axis=1)

    h0 = _mm_x_w1(x, W1)
    parts1 = _spmm(src2d, dst2d, edge_weight, h0, _ZEROS)
    m = _hidden_mm(parts1, W23)
    parts2 = _spmm(src2d, dst2d, edge_weight, m, _ZEROS)
    noise = jax.random.normal(jax.random.key(42), (N, H2), dtype=jnp.float32)
    z = _make_z(parts2, noise)
    return jnp.reshape(_decoder(z), (-1,))


# speedup vs baseline: 8.0308x; 1.0021x over previous
import functools

import numpy as np
import jax
import jax.numpy as jnp
from jax import lax
from jax.experimental import pallas as pl
from jax.experimental.pallas import tpu as pltpu
from jax.experimental.pallas import tpu_sc as plsc

N = 10000
E = 320000
F = 128
H1 = 32
H2 = 16

NC = 2
NS = 16
NW = NC * NS
SUB = 125
EW = E // NW
NSUBW = EW // SUB
RING = 4
ROWS_T = 624
TAIL = N - NS * ROWS_T


def _make_spmm():
    mesh = plsc.VectorSubcoreMesh(core_axis_name="c", subcore_axis_name="s")

    @functools.partial(
        pl.kernel,
        out_type=jax.ShapeDtypeStruct((NC, N, H1), jnp.float32),
        mesh=mesh,
        scratch_types=[
            pltpu.VMEM((NSUBW, SUB), jnp.int32),
            pltpu.VMEM((NSUBW, SUB), jnp.int32),
            pltpu.VMEM((EW,), jnp.float32),
            pltpu.VMEM((RING, SUB, H1), jnp.float32),
            pltpu.VMEM((RING, SUB, H1), jnp.float32),
            pltpu.VMEM_SHARED((N, H1), jnp.float32),
            pltpu.VMEM_SHARED((N, H1), jnp.float32),
            pltpu.SemaphoreType.DMA,
            pltpu.SemaphoreType.DMA,
        ],
        compiler_params=pltpu.CompilerParams(use_tc_tiling_on_sc=False,
                                             needs_layout_passes=False),
    )
    def spmm(src_hbm, dst_hbm, w_hbm, x_hbm, zero_hbm, out_hbm,
             src_v, dst_v, w_v, g_v, s_v, acc, x_sh, sem_g, sem_s):
        cid = lax.axis_index("c")
        sid = lax.axis_index("s")
        wid = cid * NS + sid
        base = wid * NSUBW

        pltpu.sync_copy(src_hbm.at[pl.ds(base, NSUBW)], src_v)
        pltpu.sync_copy(dst_hbm.at[pl.ds(base, NSUBW)], dst_v)
        pltpu.sync_copy(w_hbm.at[pl.ds(wid * EW, EW)], w_v)
        pltpu.sync_copy(zero_hbm.at[pl.ds(sid * ROWS_T, ROWS_T)],
                        acc.at[pl.ds(sid * ROWS_T, ROWS_T)])
        pltpu.sync_copy(x_hbm.at[pl.ds(sid * ROWS_T, ROWS_T)],
                        x_sh.at[pl.ds(sid * ROWS_T, ROWS_T)])

        @pl.when(sid == NS - 1)
        def _():
            pltpu.sync_copy(zero_hbm.at[pl.ds(NS * ROWS_T, TAIL)],
                            acc.at[pl.ds(NS * ROWS_T, TAIL)])
            pltpu.sync_copy(x_hbm.at[pl.ds(NS * ROWS_T, TAIL)],
                            x_sh.at[pl.ds(NS * ROWS_T, TAIL)])

        plsc.subcore_barrier()

        for p in range(RING - 1):
            pltpu.async_copy(x_sh.at[src_v.at[p]], g_v.at[p], sem_g)

        def process(j, carry):
            b = lax.rem(j, RING)
            pltpu.make_async_copy(zero_hbm.at[pl.ds(0, SUB)],
                                  g_v.at[b], sem_g).wait()

            @pl.when(j >= RING)
            def _():
                pltpu.make_async_copy(zero_hbm.at[pl.ds(0, SUB)],
                                      s_v.at[b], sem_s).wait()

            wbase = j * SUB
            for g in range(SUB // 16):
                w16 = w_v[pl.ds(wbase + g * 16, 16)]
                for l in range(16):
                    e = g * 16 + l
                    wv = jnp.broadcast_to(w16[l], (16,))
                    s_v[b, e, 0:16] = g_v[b, e, 0:16] * wv
                    s_v[b, e, 16:32] = g_v[b, e, 16:32] * wv

            def mul_tail(e, c2):
                wv = plsc.load_gather(w_v, [jnp.broadcast_to(wbase + e, (16,))])
                s_v[b, e, 0:16] = g_v[b, e, 0:16] * wv
                s_v[b, e, 16:32] = g_v[b, e, 16:32] * wv
                return c2

            lax.fori_loop((SUB // 16) * 16, SUB, mul_tail, 0)

            pltpu.async_copy(s_v.at[b], acc.at[dst_v.at[j]], sem_s, add=True)

            @pl.when(j + RING - 1 < NSUBW)
            def _():
                pltpu.async_copy(x_sh.at[src_v.at[j + RING - 1]],
                                 g_v.at[lax.rem(j + RING - 1, RING)], sem_g)
            return carry

        lax.fori_loop(0, NSUBW, process, 0)

        for b in range(RING):
            pltpu.make_async_copy(zero_hbm.at[pl.ds(0, SUB)],
                                  s_v.at[b], sem_s).wait()
        plsc.subcore_barrier()

        pltpu.sync_copy(acc.at[pl.ds(sid * ROWS_T, ROWS_T)],
                        out_hbm.at[cid, pl.ds(sid * ROWS_T, ROWS_T)])

        @pl.when(sid == NS - 1)
        def _():
            pltpu.sync_copy(acc.at[pl.ds(NS * ROWS_T, TAIL)],
                            out_hbm.at[cid, pl.ds(NS * ROWS_T, TAIL)])

    return spmm


_spmm = _make_spmm()

_ZEROS = np.zeros((N, H1), dtype=np.float32)


def _mm_x_w1(x, W1):
    BM = 1000

    def body(x_ref, w_ref, o_ref):
        o_ref[...] = lax.dot_general(
            x_ref[...], w_ref[...], (((1,), (0,)), ((), ())),
            preferred_element_type=jnp.float32)

    return pl.pallas_call(
        body,
        grid=(N // BM,),
        in_specs=[pl.BlockSpec((BM, F), lambda i: (i, 0)),
                  pl.BlockSpec((F, H1), lambda i: (0, 0))],
        out_specs=pl.BlockSpec((BM, H1), lambda i: (i, 0)),
        out_shape=jax.ShapeDtypeStruct((N, H1), jnp.float32),
    )(x, W1)


def _hidden_mm(parts, W23):
    BM = 1000

    def body(p_ref, w_ref, o_ref):
        h = jnp.maximum(p_ref[0] + p_ref[1], 0.0)
        o_ref[...] = lax.dot_general(
            h, w_ref[...], (((1,), (0,)), ((), ())),
            preferred_element_type=jnp.float32)

    return pl.pallas_call(
        body,
        grid=(N // BM,),
        in_specs=[pl.BlockSpec((NC, BM, H1), lambda i: (0, i, 0)),
                  pl.BlockSpec((H1, 2 * H2), lambda i: (0, 0))],
        out_specs=pl.BlockSpec((BM, 2 * H2), lambda i: (i, 0)),
        out_shape=jax.ShapeDtypeStruct((N, 2 * H2), jnp.float32),
    )(parts, W23)


def _make_z(parts, noise):
    BM = 1000

    def body(p_ref, n_ref, o_ref):
        s = p_ref[0] + p_ref[1]
        o_ref[...] = s[:, :H2] + n_ref[...] * jnp.exp(s[:, H2:])

    return pl.pallas_call(
        body,
        grid=(N // BM,),
        in_specs=[pl.BlockSpec((NC, BM, 2 * H2), lambda i: (0, i, 0)),
                  pl.BlockSpec((BM, H2), lambda i: (i, 0))],
        out_specs=pl.BlockSpec((BM, H2), lambda i: (i, 0)),
        out_shape=jax.ShapeDtypeStruct((N, H2), jnp.float32),
    )(parts, noise)


def _decoder(z):
    BM = 400

    def body(zi_ref, zj_ref, o_ref):
        o_ref[...] = lax.dot_general(
            zi_ref[...], zj_ref[...], (((1,), (1,)), ((), ())),
            preferred_element_type=jnp.float32)

    return pl.pallas_call(
        body,
        grid=(N // BM,),
        in_specs=[pl.BlockSpec((BM, H2), lambda i: (i, 0)),
                  pl.BlockSpec((N, H2), lambda i: (0, 0))],
        out_specs=pl.BlockSpec((BM, N), lambda i: (i, 0)),
        out_shape=jax.ShapeDtypeStruct((N, N), jnp.float32),
    )(z, z)


def kernel(x, edge_index, edge_weight, W1, W2, W3):
    src2d = jnp.reshape(edge_index[0], (E // SUB, SUB))
    dst2d = jnp.reshape(edge_index[1], (E // SUB, SUB))
    W23 = jnp.concatenate([W2, W3], axis=1)

    h0 = _mm_x_w1(x, W1)
    parts1 = _spmm(src2d, dst2d, edge_weight, h0, _ZEROS)
    m = _hidden_mm(parts1, W23)
    parts2 = _spmm(src2d, dst2d, edge_weight, m, _ZEROS)
    noise = jax.random.normal(jax.random.key(42), (N, H2), dtype=jnp.float32)
    z = _make_z(parts2, noise)
    return jnp.reshape(_decoder(z), (-1,))


# speedup vs baseline: 8.3261x; 1.0368x over previous
import functools

import numpy as np
import jax
import jax.numpy as jnp
from jax import lax
from jax.experimental import pallas as pl
from jax.experimental.pallas import tpu as pltpu
from jax.experimental.pallas import tpu_sc as plsc

N = 10000
E = 320000
F = 128
H1 = 32
H2 = 16

NC = 2
NS = 16
NW = NC * NS
SUB = 80
EW = E // NW
NSUBW = EW // SUB
RING = 4
ROWS_T = 624
TAIL = N - NS * ROWS_T


def _make_spmm():
    mesh = plsc.VectorSubcoreMesh(core_axis_name="c", subcore_axis_name="s")

    @functools.partial(
        pl.kernel,
        out_type=jax.ShapeDtypeStruct((NC, N, H1), jnp.float32),
        mesh=mesh,
        scratch_types=[
            pltpu.VMEM((EW,), jnp.int32),
            pltpu.VMEM((EW,), jnp.int32),
            pltpu.VMEM((EW,), jnp.float32),
            pltpu.VMEM((RING, SUB, H1), jnp.float32),
            pltpu.VMEM((RING, SUB, H1), jnp.float32),
            pltpu.VMEM_SHARED((N, H1), jnp.float32),
            pltpu.VMEM_SHARED((N, H1), jnp.float32),
            pltpu.SemaphoreType.DMA,
            pltpu.SemaphoreType.DMA,
        ],
        compiler_params=pltpu.CompilerParams(use_tc_tiling_on_sc=False,
                                             needs_layout_passes=False),
    )
    def spmm(edge_hbm, w_hbm, x_hbm, zero_hbm, out_hbm,
             src_v, dst_v, w_v, g_v, s_v, acc, x_sh, sem_g, sem_s):
        cid = lax.axis_index("c")
        sid = lax.axis_index("s")
        wid = cid * NS + sid

        pltpu.sync_copy(edge_hbm.at[0, pl.ds(wid * EW, EW)], src_v)
        pltpu.sync_copy(edge_hbm.at[1, pl.ds(wid * EW, EW)], dst_v)
        pltpu.sync_copy(w_hbm.at[pl.ds(wid * EW, EW)], w_v)
        pltpu.sync_copy(zero_hbm.at[pl.ds(sid * ROWS_T, ROWS_T)],
                        acc.at[pl.ds(sid * ROWS_T, ROWS_T)])
        pltpu.sync_copy(x_hbm.at[pl.ds(sid * ROWS_T, ROWS_T)],
                        x_sh.at[pl.ds(sid * ROWS_T, ROWS_T)])

        @pl.when(sid == NS - 1)
        def _():
            pltpu.sync_copy(zero_hbm.at[pl.ds(NS * ROWS_T, TAIL)],
                            acc.at[pl.ds(NS * ROWS_T, TAIL)])
            pltpu.sync_copy(x_hbm.at[pl.ds(NS * ROWS_T, TAIL)],
                            x_sh.at[pl.ds(NS * ROWS_T, TAIL)])

        plsc.subcore_barrier()

        for p in range(RING - 1):
            pltpu.async_copy(x_sh.at[src_v.at[pl.ds(p * SUB, SUB)]],
                             g_v.at[p], sem_g)

        def process(j, carry):
            b = lax.rem(j, RING)
            pltpu.make_async_copy(zero_hbm.at[pl.ds(0, SUB)],
                                  g_v.at[b], sem_g).wait()

            @pl.when(j >= RING)
            def _():
                pltpu.make_async_copy(zero_hbm.at[pl.ds(0, SUB)],
                                      s_v.at[b], sem_s).wait()

            wbase = j * SUB
            for g in range(SUB // 16):
                w16 = w_v[pl.ds(wbase + g * 16, 16)]
                for l in range(16):
                    e = g * 16 + l
                    wv = jnp.broadcast_to(w16[l], (16,))
                    s_v[b, e, 0:16] = g_v[b, e, 0:16] * wv
                    s_v[b, e, 16:32] = g_v[b, e, 16:32] * wv

            pltpu.async_copy(s_v.at[b], acc.at[dst_v.at[pl.ds(j * SUB, SUB)]],
                             sem_s, add=True)

            @pl.when(j + RING - 1 < NSUBW)
            def _():
                jr = j + RING - 1
                pltpu.async_copy(x_sh.at[src_v.at[pl.ds(jr * SUB, SUB)]],
                                 g_v.at[lax.rem(jr, RING)], sem_g)
            return carry

        lax.fori_loop(0, NSUBW, process, 0)

        for b in range(RING):
            pltpu.make_async_copy(zero_hbm.at[pl.ds(0, SUB)],
                                  s_v.at[b], sem_s).wait()
        plsc.subcore_barrier()

        pltpu.sync_copy(acc.at[pl.ds(sid * ROWS_T, ROWS_T)],
                        out_hbm.at[cid, pl.ds(sid * ROWS_T, ROWS_T)])

        @pl.when(sid == NS - 1)
        def _():
            pltpu.sync_copy(acc.at[pl.ds(NS * ROWS_T, TAIL)],
                            out_hbm.at[cid, pl.ds(NS * ROWS_T, TAIL)])

    return spmm


_spmm = _make_spmm()

_ZEROS = np.zeros((N, H1), dtype=np.float32)


def _mm_x_w1(x, W1):
    BM = 1000

    def body(x_ref, w_ref, o_ref):
        o_ref[...] = lax.dot_general(
            x_ref[...], w_ref[...], (((1,), (0,)), ((), ())),
            preferred_element_type=jnp.float32)

    return pl.pallas_call(
        body,
        grid=(N // BM,),
        in_specs=[pl.BlockSpec((BM, F), lambda i: (i, 0)),
                  pl.BlockSpec((F, H1), lambda i: (0, 0))],
        out_specs=pl.BlockSpec((BM, H1), lambda i: (i, 0)),
        out_shape=jax.ShapeDtypeStruct((N, H1), jnp.float32),
    )(x, W1)


def _hidden_mm(parts, W23):
    BM = 1000

    def body(p_ref, w_ref, o_ref):
        h = jnp.maximum(p_ref[0] + p_ref[1], 0.0)
        o_ref[...] = lax.dot_general(
            h, w_ref[...], (((1,), (0,)), ((), ())),
            preferred_element_type=jnp.float32)

    return pl.pallas_call(
        body,
        grid=(N // BM,),
        in_specs=[pl.BlockSpec((NC, BM, H1), lambda i: (0, i, 0)),
                  pl.BlockSpec((H1, 2 * H2), lambda i: (0, 0))],
        out_specs=pl.BlockSpec((BM, 2 * H2), lambda i: (i, 0)),
        out_shape=jax.ShapeDtypeStruct((N, 2 * H2), jnp.float32),
    )(parts, W23)


def _decoder(parts, noise):
    BM = 400

    def body(p_ref, n_ref, o_ref, z_ref):
        i = pl.program_id(0)

        @pl.when(i == 0)
        def _():
            s = p_ref[0] + p_ref[1]
            z_ref[...] = s[:, :H2] + n_ref[...] * jnp.exp(s[:, H2:])

        o_ref[...] = lax.dot_general(
            z_ref[pl.ds(i * BM, BM), :], z_ref[...],
            (((1,), (1,)), ((), ())), preferred_element_type=jnp.float32)

    return pl.pallas_call(
        body,
        grid=(N // BM,),
        in_specs=[pl.BlockSpec((NC, N, 2 * H2), lambda i: (0, 0, 0)),
                  pl.BlockSpec((N, H2), lambda i: (0, 0))],
        out_specs=pl.BlockSpec((BM, N), lambda i: (i, 0)),
        out_shape=jax.ShapeDtypeStruct((N, N), jnp.float32),
        scratch_shapes=[pltpu.VMEM((N, H2), jnp.float32)],
    )(parts, noise)


def kernel(x, edge_index, edge_weight, W1, W2, W3):
    W23 = jnp.concatenate([W2, W3], axis=1)

    h0 = _mm_x_w1(x, W1)
    parts1 = _spmm(edge_index, edge_weight, h0, _ZEROS)
    m = _hidden_mm(parts1, W23)
    parts2 = _spmm(edge_index, edge_weight, m, _ZEROS)
    noise = jax.random.normal(jax.random.key(42), (N, H2), dtype=jnp.float32)
    return jnp.reshape(_decoder(parts2, noise), (-1,))


# speedup vs baseline: 8.3839x; 1.0069x over previous
import functools

import numpy as np
import jax
import jax.numpy as jnp
from jax import lax
from jax.experimental import pallas as pl
from jax.experimental.pallas import tpu as pltpu
from jax.experimental.pallas import tpu_sc as plsc

N = 10000
E = 320000
F = 128
H1 = 32
H2 = 16

NC = 2
NS = 16
NW = NC * NS
SUB = 80
EW = E // NW
NSUBW = EW // SUB
RING = 6
ROWS_T = 624
TAIL = N - NS * ROWS_T


def _make_spmm():
    mesh = plsc.VectorSubcoreMesh(core_axis_name="c", subcore_axis_name="s")

    @functools.partial(
        pl.kernel,
        out_type=jax.ShapeDtypeStruct((NC, N, H1), jnp.float32),
        mesh=mesh,
        scratch_types=[
            pltpu.VMEM((EW,), jnp.int32),
            pltpu.VMEM((EW,), jnp.int32),
            pltpu.VMEM((EW,), jnp.float32),
            pltpu.VMEM((RING, SUB, H1), jnp.float32),
            pltpu.VMEM((RING, SUB, H1), jnp.float32),
            pltpu.VMEM_SHARED((N, H1), jnp.float32),
            pltpu.VMEM_SHARED((N, H1), jnp.float32),
            pltpu.SemaphoreType.DMA,
            pltpu.SemaphoreType.DMA,
        ],
        compiler_params=pltpu.CompilerParams(use_tc_tiling_on_sc=False,
                                             needs_layout_passes=False),
    )
    def spmm(edge_hbm, w_hbm, x_hbm, zero_hbm, out_hbm,
             src_v, dst_v, w_v, g_v, s_v, acc, x_sh, sem_g, sem_s):
        cid = lax.axis_index("c")
        sid = lax.axis_index("s")
        wid = cid * NS + sid

        pltpu.sync_copy(edge_hbm.at[0, pl.ds(wid * EW, EW)], src_v)
        pltpu.sync_copy(edge_hbm.at[1, pl.ds(wid * EW, EW)], dst_v)
        pltpu.sync_copy(w_hbm.at[pl.ds(wid * EW, EW)], w_v)
        pltpu.sync_copy(zero_hbm.at[pl.ds(sid * ROWS_T, ROWS_T)],
                        acc.at[pl.ds(sid * ROWS_T, ROWS_T)])
        pltpu.sync_copy(x_hbm.at[pl.ds(sid * ROWS_T, ROWS_T)],
                        x_sh.at[pl.ds(sid * ROWS_T, ROWS_T)])

        @pl.when(sid == NS - 1)
        def _():
            pltpu.sync_copy(zero_hbm.at[pl.ds(NS * ROWS_T, TAIL)],
                            acc.at[pl.ds(NS * ROWS_T, TAIL)])
            pltpu.sync_copy(x_hbm.at[pl.ds(NS * ROWS_T, TAIL)],
                            x_sh.at[pl.ds(NS * ROWS_T, TAIL)])

        plsc.subcore_barrier()

        for p in range(RING - 1):
            pltpu.async_copy(x_sh.at[src_v.at[pl.ds(p * SUB, SUB)]],
                             g_v.at[p], sem_g)

        def process(j, carry):
            b = lax.rem(j, RING)
            pltpu.make_async_copy(zero_hbm.at[pl.ds(0, SUB)],
                                  g_v.at[b], sem_g).wait()

            @pl.when(j >= RING)
            def _():
                pltpu.make_async_copy(zero_hbm.at[pl.ds(0, SUB)],
                                      s_v.at[b], sem_s).wait()

            wbase = j * SUB
            for g in range(SUB // 16):
                w16 = w_v[pl.ds(wbase + g * 16, 16)]
                for l in range(16):
                    e = g * 16 + l
                    wv = jnp.broadcast_to(w16[l], (16,))
                    s_v[b, e, 0:16] = g_v[b, e, 0:16] * wv
                    s_v[b, e, 16:32] = g_v[b, e, 16:32] * wv

            pltpu.async_copy(s_v.at[b], acc.at[dst_v.at[pl.ds(j * SUB, SUB)]],
                             sem_s, add=True)

            @pl.when(j + RING - 1 < NSUBW)
            def _():
                jr = j + RING - 1
                pltpu.async_copy(x_sh.at[src_v.at[pl.ds(jr * SUB, SUB)]],
                                 g_v.at[lax.rem(jr, RING)], sem_g)
            return carry

        lax.fori_loop(0, NSUBW, process, 0)

        for b in range(RING):
            pltpu.make_async_copy(zero_hbm.at[pl.ds(0, SUB)],
                                  s_v.at[b], sem_s).wait()
        plsc.subcore_barrier()

        pltpu.sync_copy(acc.at[pl.ds(sid * ROWS_T, ROWS_T)],
                        out_hbm.at[cid, pl.ds(sid * ROWS_T, ROWS_T)])

        @pl.when(sid == NS - 1)
        def _():
            pltpu.sync_copy(acc.at[pl.ds(NS * ROWS_T, TAIL)],
                            out_hbm.at[cid, pl.ds(NS * ROWS_T, TAIL)])

    return spmm


_spmm = _make_spmm()

_ZEROS = np.zeros((N, H1), dtype=np.float32)


def _mm_x_w1(x, W1):
    BM = 2000

    def body(x_ref, w_ref, o_ref):
        o_ref[...] = lax.dot_general(
            x_ref[...], w_ref[...], (((1,), (0,)), ((), ())),
            preferred_element_type=jnp.float32)

    return pl.pallas_call(
        body,
        grid=(N // BM,),
        in_specs=[pl.BlockSpec((BM, F), lambda i: (i, 0)),
                  pl.BlockSpec((F, H1), lambda i: (0, 0))],
        out_specs=pl.BlockSpec((BM, H1), lambda i: (i, 0)),
        out_shape=jax.ShapeDtypeStruct((N, H1), jnp.float32),
    )(x, W1)


def _hidden_mm(parts, W23):
    BM = 2000

    def body(p_ref, w_ref, o_ref):
        h = jnp.maximum(p_ref[0] + p_ref[1], 0.0)
        o_ref[...] = lax.dot_general(
            h, w_ref[...], (((1,), (0,)), ((), ())),
            preferred_element_type=jnp.float32)

    return pl.pallas_call(
        body,
        grid=(N // BM,),
        in_specs=[pl.BlockSpec((NC, BM, H1), lambda i: (0, i, 0)),
                  pl.BlockSpec((H1, 2 * H2), lambda i: (0, 0))],
        out_specs=pl.BlockSpec((BM, 2 * H2), lambda i: (i, 0)),
        out_shape=jax.ShapeDtypeStruct((N, 2 * H2), jnp.float32),
    )(parts, W23)


def _decoder(parts, noise):
    BM = 400

    def body(p_ref, n_ref, o_ref, z_ref):
        i = pl.program_id(0)

        @pl.when(i == 0)
        def _():
            s = p_ref[0] + p_ref[1]
            z_ref[...] = s[:, :H2] + n_ref[...] * jnp.exp(s[:, H2:])

        o_ref[...] = lax.dot_general(
            z_ref[pl.ds(i * BM, BM), :], z_ref[...],
            (((1,), (1,)), ((), ())), preferred_element_type=jnp.float32)

    return pl.pallas_call(
        body,
        grid=(N // BM,),
        in_specs=[pl.BlockSpec((NC, N, 2 * H2), lambda i: (0, 0, 0)),
                  pl.BlockSpec((N, H2), lambda i: (0, 0))],
        out_specs=pl.BlockSpec((BM, N), lambda i: (i, 0)),
        out_shape=jax.ShapeDtypeStruct((N, N), jnp.float32),
        scratch_shapes=[pltpu.VMEM((N, H2), jnp.float32)],
    )(parts, noise)


def kernel(x, edge_index, edge_weight, W1, W2, W3):
    W23 = jnp.concatenate([W2, W3], axis=1)

    h0 = _mm_x_w1(x, W1)
    parts1 = _spmm(edge_index, edge_weight, h0, _ZEROS)
    m = _hidden_mm(parts1, W23)
    parts2 = _spmm(edge_index, edge_weight, m, _ZEROS)
    noise = jax.random.normal(jax.random.key(42), (N, H2), dtype=jnp.float32)
    return jnp.reshape(_decoder(parts2, noise), (-1,))
